# baseline (speedup 1.0000x reference)
import functools

import jax
import jax.numpy as jnp
from jax import lax
from jax.experimental import pallas as pl
from jax.experimental.pallas import tpu as pltpu
from jax.experimental.pallas import tpu_sc as plsc

_NUM_STOCKS = 1000000
_D = 16
_BATCH = 16384
_HIST = 20

_B = _BATCH * _HIST
_NW = 32
_CBLK = 128
_NBLK = _BATCH // _CBLK
_BLK_PER_W = _NBLK // _NW
_ROWS = _CBLK * _HIST
_NTILE = _HIST * 2

_NCOLS = 7813
_PAD_STOCKS = _NCOLS * 128
_DEPTH = 4

_mesh = plsc.VectorSubcoreMesh(core_axis_name="c", subcore_axis_name="s")


@functools.partial(
    pl.kernel,
    mesh=_mesh,
    out_type=jax.ShapeDtypeStruct((_PAD_STOCKS * _D,), jnp.float32),
    scratch_types=[
        [pltpu.VMEM((_D, 128), jnp.float32) for _ in range(_DEPTH)],
        [pltpu.VMEM((2048,), jnp.float32) for _ in range(_DEPTH)],
        [pltpu.SemaphoreType.DMA for _ in range(_DEPTH)],
        [pltpu.SemaphoreType.DMA for _ in range(_DEPTH)],
    ],
    compiler_params=pltpu.CompilerParams(
        use_tc_tiling_on_sc=True, needs_layout_passes=False),
)
def _detile_kernel(tabt_hbm, out_hbm, blks, stgs, isems, osems):
    wid = lax.axis_index("s") * 2 + lax.axis_index("c")
    iota16 = lax.iota(jnp.int32, 16) * 16
    start = wid * 244 + jnp.minimum(wid, 5)
    n = 244 + (wid < 5).astype(jnp.int32)

    def in_slice(c):
        return tabt_hbm.at[:, pl.ds(c * 128, 128)]

    def out_slice(c):
        return out_hbm.at[pl.ds(c * 2048, 2048)]

    for b in range(_DEPTH):
        pltpu.async_copy(in_slice(start + b), blks[b], isems[b])

    def jj_body(jj, carry):
        for b in range(_DEPTH):
            j = _DEPTH * jj + b
            c = start + j

            @pl.when((j >= _DEPTH) & (j - _DEPTH < n))
            def _():
                pltpu.make_async_copy(stgs[b], out_slice(c - _DEPTH),
                                      osems[b]).wait()

            @pl.when(j < n)
            def _():
                pltpu.make_async_copy(in_slice(c), blks[b], isems[b]).wait()
                for d in range(_D):
                    vs = [blks[b][d, pl.ds(16 * q, 16)] for q in range(8)]
                    for q in range(8):
                        plsc.store_scatter(
                            stgs[b], [iota16 + (256 * q + d)], vs[q])
                pltpu.async_copy(stgs[b], out_slice(c), osems[b])

            @pl.when(j + _DEPTH < n)
            def _():
                pltpu.async_copy(in_slice(c + _DEPTH), blks[b], isems[b])

        return carry

    lax.fori_loop(0, 62, jj_body, 0)

    @pl.when(n == 245)
    def _():
        pltpu.make_async_copy(stgs[244 % _DEPTH], out_slice(start + 244),
                              osems[244 % _DEPTH]).wait()


@functools.partial(
    pl.kernel,
    mesh=_mesh,
    out_type=jax.ShapeDtypeStruct((_NTILE, _NBLK, 1024), jnp.float32),
    scratch_types=[
        [pltpu.VMEM((_ROWS,), jnp.int32) for _ in range(2)],
        [pltpu.VMEM((_ROWS, _D), jnp.float32) for _ in range(2)],
        pltpu.VMEM((_NTILE * 1024,), jnp.float32),
        [pltpu.SemaphoreType.DMA for _ in range(2)],
        pltpu.SemaphoreType.DMA,
    ],
    compiler_params=pltpu.CompilerParams(
        use_tc_tiling_on_sc=False, needs_layout_passes=False),
)
def _gather_kernel(idx_hbm, table_hbm, out_hbm, idxs, rows, outs_v,
                   gsems, osem):
    wid = lax.axis_index("s") * 2 + lax.axis_index("c")
    iota = lax.iota(jnp.int32, 16)
    dvec = (iota // 8) * 1024 + (iota % 8) * 128
    dvecs = [dvec + 2048 * h for h in range(_HIST)]
    c0 = wid * _BLK_PER_W

    def drain_outs(c):
        def out_drain(j, carry2):
            pltpu.make_async_copy(outs_v.at[pl.ds(j * 1024, 1024)],
                                  out_hbm.at[j, c], osem).wait()
            return carry2

        lax.fori_loop(0, _NTILE, out_drain, 0)

    pltpu.sync_copy(idx_hbm.at[pl.ds(c0 * _ROWS, _ROWS)], idxs[0])
    gather = [pltpu.async_copy(table_hbm.at[idxs[0]], rows[0], gsems[0]),
              None]
    for cb in range(_BLK_PER_W):
        b = cb % 2
        c = c0 + cb
        if cb + 1 < _BLK_PER_W:
            pltpu.sync_copy(idx_hbm.at[pl.ds((c + 1) * _ROWS, _ROWS)],
                            idxs[1 - b])
            gather[1 - b] = pltpu.async_copy(
                table_hbm.at[idxs[1 - b]], rows[1 - b], gsems[1 - b])
        gather[b].wait()
        if cb > 0:
            drain_outs(c - 1)

        def row_body(g, carry2, b=b):
            base = g * 4 * _HIST
            for u in range(4):
                bl = g * 4 + u
                for h0 in range(0, _HIST, 10):
                    vs = [rows[b][base + (u * _HIST + h0 + i)]
                          for i in range(10)]
                    for i in range(10):
                        plsc.store_scatter(
                            outs_v, [dvecs[h0 + i] + bl], vs[i])
            return carry2

        lax.fori_loop(0, _CBLK // 4, row_body, 0)

        def out_start(j, carry2, c=c):
            pltpu.async_copy(outs_v.at[pl.ds(j * 1024, 1024)],
                             out_hbm.at[j, c], osem)
            return carry2

        lax.fori_loop(0, _NTILE, out_start, 0)
    drain_outs(c0 + _BLK_PER_W - 1)


def kernel(batch_x, embed_table):
    idx = batch_x.reshape(_B).astype(jnp.int32)
    table_lin = _detile_kernel(embed_table.T).reshape(_PAD_STOCKS, _D)
    out4 = _gather_kernel(idx, table_lin)
    out = out4.reshape(_HIST, 2, _NBLK, 8, 128)
    out = out.transpose(2, 4, 0, 1, 3)
    return out.reshape(_BATCH, _HIST, _D)


# speedup vs baseline: 1.0453x; 1.0453x over previous
import functools

import jax
import jax.numpy as jnp
from jax import lax
from jax.experimental import pallas as pl
from jax.experimental.pallas import tpu as pltpu
from jax.experimental.pallas import tpu_sc as plsc

_NUM_STOCKS = 1000000
_D = 16
_BATCH = 16384
_HIST = 20

_B = _BATCH * _HIST
_NW = 32
_CBLK = 128
_NBLK = _BATCH // _CBLK
_BLK_PER_W = _NBLK // _NW
_ROWS = _CBLK * _HIST
_NTILE = _HIST * 2

_NCOLS = 7813
_PAD_STOCKS = _NCOLS * 128
_DEPTH = 4

_mesh = plsc.VectorSubcoreMesh(core_axis_name="c", subcore_axis_name="s")


@functools.partial(
    pl.kernel,
    mesh=_mesh,
    out_type=jax.ShapeDtypeStruct((_PAD_STOCKS * _D,), jnp.float32),
    scratch_types=[
        [pltpu.VMEM((_D, 128), jnp.float32) for _ in range(_DEPTH)],
        [pltpu.VMEM((2048,), jnp.float32) for _ in range(_DEPTH)],
        [pltpu.SemaphoreType.DMA for _ in range(_DEPTH)],
        [pltpu.SemaphoreType.DMA for _ in range(_DEPTH)],
    ],
    compiler_params=pltpu.CompilerParams(
        use_tc_tiling_on_sc=True, needs_layout_passes=False),
)
def _detile_kernel(tabt_hbm, out_hbm, blks, stgs, isems, osems):
    wid = lax.axis_index("s") * 2 + lax.axis_index("c")
    iota16 = lax.iota(jnp.int32, 16) * 16
    start = wid * 244 + jnp.minimum(wid, 5)
    n = 244 + (wid < 5).astype(jnp.int32)

    def in_slice(c):
        return tabt_hbm.at[:, pl.ds(c * 128, 128)]

    def out_slice(c):
        return out_hbm.at[pl.ds(c * 2048, 2048)]

    for b in range(_DEPTH):
        pltpu.async_copy(in_slice(start + b), blks[b], isems[b])

    def jj_body(jj, carry):
        for b in range(_DEPTH):
            j = _DEPTH * jj + b
            c = start + j

            @pl.when((j >= _DEPTH) & (j - _DEPTH < n))
            def _():
                pltpu.make_async_copy(stgs[b], out_slice(c - _DEPTH),
                                      osems[b]).wait()

            @pl.when(j < n)
            def _():
                pltpu.make_async_copy(in_slice(c), blks[b], isems[b]).wait()
                for d in range(_D):
                    for q in range(8):
                        vals = blks[b][d, pl.ds(16 * q, 16)]
                        plsc.store_scatter(
                            stgs[b], [iota16 + (256 * q + d)], vals)
                pltpu.async_copy(stgs[b], out_slice(c), osems[b])

            @pl.when(j + _DEPTH < n)
            def _():
                pltpu.async_copy(in_slice(c + _DEPTH), blks[b], isems[b])

        return carry

    lax.fori_loop(0, 62, jj_body, 0)

    @pl.when(n == 245)
    def _():
        pltpu.make_async_copy(stgs[244 % _DEPTH], out_slice(start + 244),
                              osems[244 % _DEPTH]).wait()


@functools.partial(
    pl.kernel,
    mesh=_mesh,
    out_type=jax.ShapeDtypeStruct((_NTILE, _NBLK, 1024), jnp.float32),
    scratch_types=[
        [pltpu.VMEM((_ROWS,), jnp.int32) for _ in range(2)],
        [pltpu.VMEM((_ROWS, _D), jnp.float32) for _ in range(2)],
        pltpu.VMEM((_NTILE * 1024,), jnp.float32),
        [pltpu.SemaphoreType.DMA for _ in range(2)],
        pltpu.SemaphoreType.DMA,
    ],
    compiler_params=pltpu.CompilerParams(
        use_tc_tiling_on_sc=False, needs_layout_passes=False),
)
def _gather_kernel(idx_hbm, table_hbm, out_hbm, idxs, rows, outs_v,
                   gsems, osem):
    wid = lax.axis_index("s") * 2 + lax.axis_index("c")
    iota = lax.iota(jnp.int32, 16)
    dvec = (iota // 8) * 1024 + (iota % 8) * 128
    dvecs = [dvec + 2048 * h for h in range(_HIST)]
    c0 = wid * _BLK_PER_W

    def drain_outs(c):
        def out_drain(j, carry2):
            pltpu.make_async_copy(outs_v.at[pl.ds(j * 1024, 1024)],
                                  out_hbm.at[j, c], osem).wait()
            return carry2

        lax.fori_loop(0, _NTILE, out_drain, 0)

    pltpu.sync_copy(idx_hbm.at[pl.ds(c0 * _ROWS, _ROWS)], idxs[0])
    gather = [pltpu.async_copy(table_hbm.at[idxs[0]], rows[0], gsems[0]),
              None]
    for cb in range(_BLK_PER_W):
        b = cb % 2
        c = c0 + cb
        if cb + 1 < _BLK_PER_W:
            pltpu.sync_copy(idx_hbm.at[pl.ds((c + 1) * _ROWS, _ROWS)],
                            idxs[1 - b])
            gather[1 - b] = pltpu.async_copy(
                table_hbm.at[idxs[1 - b]], rows[1 - b], gsems[1 - b])
        gather[b].wait()
        if cb > 0:
            drain_outs(c - 1)

        def row_body(g, carry2, b=b):
            base = g * 4 * _HIST
            for u in range(4):
                bl = g * 4 + u
                for h0 in range(0, _HIST, 10):
                    vs = [rows[b][base + (u * _HIST + h0 + i)]
                          for i in range(10)]
                    for i in range(10):
                        plsc.store_scatter(
                            outs_v, [dvecs[h0 + i] + bl], vs[i])
            return carry2

        lax.fori_loop(0, _CBLK // 4, row_body, 0)

        def out_start(j, carry2, c=c):
            pltpu.async_copy(outs_v.at[pl.ds(j * 1024, 1024)],
                             out_hbm.at[j, c], osem)
            return carry2

        lax.fori_loop(0, _NTILE, out_start, 0)
    drain_outs(c0 + _BLK_PER_W - 1)


def kernel(batch_x, embed_table):
    idx = batch_x.reshape(_B).astype(jnp.int32)
    table_lin = _detile_kernel(embed_table.T).reshape(_PAD_STOCKS, _D)
    out4 = _gather_kernel(idx, table_lin)
    out = out4.reshape(_HIST, 2, _NBLK, 8, 128)
    out = out.transpose(2, 4, 0, 1, 3)
    return out.reshape(_BATCH, _HIST, _D)
